# baseline (speedup 1.0000x reference)
import functools

import jax
import jax.numpy as jnp
from jax import lax
from jax.experimental import pallas as pl
from jax.experimental.pallas import tpu as pltpu
from jax.experimental.pallas import tpu_sc as plsc

_D = 64
_G = 128
_U = 8

_info = plsc.get_sparse_core_info()
_NC = _info.num_cores
_NS = _info.num_subcores
_NW = _NC * _NS


def _make_gather(n_rows: int):
    rows_per_w = n_rows // _NW
    n_iter = rows_per_w // _U
    mesh = plsc.VectorSubcoreMesh(core_axis_name="c", subcore_axis_name="s")

    @functools.partial(
        pl.kernel,
        mesh=mesh,
        out_type=jax.ShapeDtypeStruct((n_rows * _G, _D), jnp.float32),
        scratch_types=[
            pltpu.VMEM((_U, _G), jnp.int32),
            pltpu.VMEM((_U * _G, _D), jnp.float32),
            pltpu.SemaphoreType.DMA,
        ],
        compiler_params=pltpu.CompilerParams(use_tc_tiling_on_sc=False),
    )
    def k(table_hbm, idx_hbm, out_hbm, idx_v, rows_v, sem):
        wid = lax.axis_index("s") * _NC + lax.axis_index("c")
        w_row0 = wid * rows_per_w

        def body(g, carry):
            row0 = w_row0 + g * _U
            pltpu.sync_copy(idx_hbm.at[pl.ds(row0, _U)], idx_v)
            descs = [
                pltpu.async_copy(
                    table_hbm.at[idx_v.at[j]],
                    rows_v.at[pl.ds(j * _G, _G)],
                    sem,
                )
                for j in range(_U)
            ]
            for d in descs:
                d.wait()
            pltpu.sync_copy(rows_v, out_hbm.at[pl.ds(row0 * _G, _U * _G)])
            return carry

        lax.fori_loop(0, n_iter, body, 0)

    return k


def kernel(x, table):
    b, s = x.shape
    x2d = x.reshape(-1, _G)
    out = _make_gather(x2d.shape[0])(table, x2d)
    return out.reshape(b, s, _D)


# speedup vs baseline: 1.0172x; 1.0172x over previous
import functools

import jax
import jax.numpy as jnp
from jax import lax
from jax.experimental import pallas as pl
from jax.experimental.pallas import tpu as pltpu
from jax.experimental.pallas import tpu_sc as plsc

_D = 64
_G = 128
_U = 5
_NBUF = 2

_info = plsc.get_sparse_core_info()
_NC = _info.num_cores
_NS = _info.num_subcores
_NW = _NC * _NS


def _make_gather(n_rows: int):
    rows_per_w = n_rows // _NW
    n_chunks = rows_per_w // _U
    assert n_chunks % _NBUF == 0
    mesh = plsc.VectorSubcoreMesh(core_axis_name="c", subcore_axis_name="s")

    @functools.partial(
        pl.kernel,
        mesh=mesh,
        out_type=jax.ShapeDtypeStruct((n_rows * _G, _D), jnp.float32),
        scratch_types=[
            pltpu.VMEM((rows_per_w, _G), jnp.int32),
            pltpu.VMEM((_NBUF, _U * _G, _D), jnp.float32),
            pltpu.SemaphoreType.DMA((_NBUF,)),
            pltpu.SemaphoreType.DMA((_NBUF,)),
        ],
        compiler_params=pltpu.CompilerParams(use_tc_tiling_on_sc=False),
    )
    def k(table_hbm, idx_hbm, out_hbm, idx_v, rows_v, gsem, osem):
        wid = lax.axis_index("s") * _NC + lax.axis_index("c")
        w_row0 = wid * rows_per_w

        def fire_gathers(c, buf):
            for j in range(_U):
                pltpu.async_copy(
                    table_hbm.at[idx_v.at[c * _U + j]],
                    rows_v.at[buf, pl.ds(j * _G, _G)],
                    gsem.at[buf],
                )

        def drain_gathers(buf):
            for j in range(_U):
                pltpu.make_async_copy(
                    table_hbm.at[idx_v.at[0]],
                    rows_v.at[buf, pl.ds(j * _G, _G)],
                    gsem.at[buf],
                ).wait()

        def fire_write(c, buf):
            pltpu.async_copy(
                rows_v.at[buf],
                out_hbm.at[pl.ds((w_row0 + c * _U) * _G, _U * _G)],
                osem.at[buf],
            )

        def drain_write(buf):
            pltpu.make_async_copy(
                rows_v.at[buf],
                out_hbm.at[pl.ds(w_row0 * _G, _U * _G)],
                osem.at[buf],
            ).wait()

        pltpu.sync_copy(idx_hbm.at[pl.ds(w_row0, rows_per_w)], idx_v)
        fire_gathers(0, 0)

        @pl.loop(0, n_chunks, step=_NBUF)
        def trip(g):
            @pl.when(g > 0)
            def _():
                drain_write(1)
            fire_gathers(g + 1, 1)
            drain_gathers(0)
            fire_write(g, 0)

            @pl.when(g < n_chunks - _NBUF)
            def _():
                drain_write(0)
                fire_gathers(g + 2, 0)
            drain_gathers(1)
            fire_write(g + 1, 1)

        drain_write(0)
        drain_write(1)

    return k


def kernel(x, table):
    b, s = x.shape
    x2d = x.reshape(-1, _G)
    out = _make_gather(x2d.shape[0])(table, x2d)
    return out.reshape(b, s, _D)
